# baseline (speedup 1.0000x reference)
import functools

import jax
import jax.numpy as jnp
from jax import lax
from jax.experimental import pallas as pl
from jax.experimental.pallas import tpu as pltpu
from jax.experimental.pallas import tpu_sc as plsc

_B, _L, _V = 1024, 50, 100000
_LP = 64
_NW = 32
_RPW = _B // _NW
_HC = _V // 2
_TOKROWS = (_RPW * _LP) // 128


def _rsqrt_vec(x):
  i = lax.bitcast_convert_type(x, jnp.int32)
  i = jnp.int32(0x5F3759DF) - lax.shift_right_logical(i, 1)
  y = lax.bitcast_convert_type(i, jnp.float32)
  for _ in range(4):
    y = y * (1.5 - 0.5 * x * y * y)
  return y


def _sc_body(tok_hbm, idf_hbm, out_hbm,
             tok2d, w2d, tb, wb, buf0, buf1, gsem, zsem0, zsem1):
  wid = lax.axis_index("s") * 2 + lax.axis_index("c")

  pltpu.sync_copy(tok_hbm.at[wid], tok2d)

  gds = [pltpu.async_copy(idf_hbm.at[tok2d.at[j]], w2d.at[j], gsem)
         for j in range(_TOKROWS)]

  zvec = jnp.zeros((16,), jnp.float32)

  def _zz(buf):
    def body(i, c):
      buf[pl.ds(i * 16, 16)] = zvec
      return c
    lax.fori_loop(0, _HC // 16, body, 0)
  _zz(buf0)
  _zz(buf1)

  for d in gds:
    d.wait()

  lane = lax.iota(jnp.int32, 16)
  padmask = lane < (_L - 3 * 16)

  bufs = (buf0, buf1)
  zsems = (zsem0, zsem1)
  zdesc = [None, None]
  prev_ts = None
  for r in range(_RPW):
    ts, ws = [], []
    for i in range(4):
      off = r * _LP + i * 16
      t_i = tok2d[off // 128, pl.ds(off % 128, 16)]
      w_i = w2d[off // 128, pl.ds(off % 128, 16)]
      if i == 3:
        w_i = jnp.where(padmask, w_i, 0.0)
      tb[i, :] = t_i
      wb[i, :] = w_i
      ts.append(t_i)
      ws.append(w_i)

    def _rot(s, cacc):
      idxs = (lane + s) & 15
      cacc = list(cacc)
      for j in range(4):
        bj = jnp.broadcast_to(jnp.int32(j), (16,))
        tsv = plsc.load_gather(tb, [bj, idxs])
        wsv = plsc.load_gather(wb, [bj, idxs])
        for i in range(4):
          cacc[i] = cacc[i] + jnp.where(ts[i] == tsv, wsv, 0.0)
      return tuple(cacc)

    cs = lax.fori_loop(0, 16, _rot, (zvec, zvec, zvec, zvec))

    acc = cs[0] * ws[0] + cs[1] * ws[1] + cs[2] * ws[2] + cs[3] * ws[3]
    normsq = jnp.sum(acc)
    nv = jnp.broadcast_to(normsq, (16,))
    s = nv * _rsqrt_vec(jnp.maximum(nv, 1e-37))
    scale = 1.0 / (s + 1e-10)
    wf = [c_i * scale for c_i in cs]

    for c in range(2):
      buf = bufs[c]
      lo = jnp.int32(c * _HC)
      if zdesc[c] is not None:
        zdesc[c].wait()
        for i in range(4):
          pt = prev_ts[i]
          m = (pt >= lo) & (pt < lo + _HC)
          plsc.store_scatter(buf, [pt - lo], zvec, mask=m)
      for i in range(4):
        m = (ts[i] >= lo) & (ts[i] < lo + _HC)
        plsc.store_scatter(buf, [ts[i] - lo], wf[i], mask=m)
      base = pl.multiple_of((wid * _RPW + r) * _V + c * _HC, 8)
      zdesc[c] = pltpu.async_copy(buf, out_hbm.at[pl.ds(base, _HC)], zsems[c])
    prev_ts = ts

  zdesc[0].wait()
  zdesc[1].wait()


_sc_bow = functools.partial(
    pl.kernel,
    out_type=jax.ShapeDtypeStruct((_B * _V,), jnp.float32),
    mesh=plsc.VectorSubcoreMesh(core_axis_name="c", subcore_axis_name="s"),
    compiler_params=pltpu.CompilerParams(needs_layout_passes=False),
    scratch_types=[
        pltpu.VMEM((_TOKROWS, 128), jnp.int32),
        pltpu.VMEM((_TOKROWS, 128), jnp.float32),
        pltpu.VMEM((4, 16), jnp.int32),
        pltpu.VMEM((4, 16), jnp.float32),
        pltpu.VMEM((_HC,), jnp.float32),
        pltpu.VMEM((_HC,), jnp.float32),
        pltpu.SemaphoreType.DMA,
        pltpu.SemaphoreType.DMA,
        pltpu.SemaphoreType.DMA,
    ],
)(_sc_body)


@jax.jit
def kernel(tokens, idf):
  B, L = tokens.shape
  (V,) = idf.shape
  tokens = tokens.astype(jnp.int32)
  pad = jnp.broadcast_to(tokens[:, :1], (B, _LP - L))
  tokp = jnp.concatenate([tokens, pad], axis=1)
  tokp = tokp.reshape(_NW, _TOKROWS, 128)
  out = _sc_bow(tokp, idf.astype(jnp.float32))
  return out.reshape(B, V)


# speedup vs baseline: 1.9514x; 1.9514x over previous
import functools

import jax
import jax.numpy as jnp
from jax import lax
from jax.experimental import pallas as pl
from jax.experimental.pallas import tpu as pltpu
from jax.experimental.pallas import tpu_sc as plsc

_B, _L, _V = 1024, 50, 100000
_LP = 64
_NW = 32
_RPW = _B // _NW
_NS = _RPW // 8
_CCH = 6144
_NCH = 16
_TAIL = _V - _NCH * _CCH
_TOKROWS = (_RPW * _LP) // 128


def _rsqrt_vec(x):
  i = lax.bitcast_convert_type(x, jnp.int32)
  i = jnp.int32(0x5F3759DF) - lax.shift_right_logical(i, 1)
  y = lax.bitcast_convert_type(i, jnp.float32)
  for _ in range(4):
    y = y * (1.5 - 0.5 * x * y * y)
  return y


def _sc_body(tok_hbm, tokflat_hbm, idf_hbm, out_hbm,
             tok2d, tokflat, w2d, wflat, wfstage, tb, wb, bufA, bufB, bufT,
             gsem, semA, semB, semT):
  wid = lax.axis_index("s") * 2 + lax.axis_index("c")

  pltpu.sync_copy(tok_hbm.at[wid], tok2d)
  pltpu.sync_copy(tokflat_hbm.at[wid], tokflat)

  gds = [pltpu.async_copy(idf_hbm.at[tok2d.at[j]], w2d.at[j], gsem)
         for j in range(_TOKROWS)]

  zvec = jnp.zeros((16,), jnp.float32)
  for buf, cols in ((bufA, _CCH), (bufB, _CCH), (bufT, _TAIL)):
    for r in range(8):
      def _z(i, c, buf=buf, r=r):
        buf[r, pl.ds(i * 16, 16)] = zvec
        return c
      lax.fori_loop(0, cols // 16, _z, 0)

  for d in gds:
    d.wait()

  for j in range(_TOKROWS):
    for i in range(8):
      wflat[pl.ds(j * 128 + i * 16, 16)] = w2d[j, pl.ds(i * 16, 16)]

  lane = lax.iota(jnp.int32, 16)
  padmask = lane < (_L - 3 * 16)

  def _chunk_buf(c):
    if c == _NCH:
      return bufT, semT, _NCH * _CCH, _TAIL
    return ((bufA, semA) if c % 2 == 0 else (bufB, semB)) + (c * _CCH, _CCH)

  def _reset_rows(buf, g, plo, psz):
    def body(r8, carry):
      rowv = jnp.broadcast_to(r8, (16,)).astype(jnp.int32)
      tbase = g * 8 * _LP + r8 * _LP
      for i in range(4):
        t = tokflat[pl.ds(tbase + i * 16, 16)]
        m = (t >= plo) & (t < plo + psz)
        plsc.store_scatter(buf, [rowv, t - plo], zvec, mask=m)
      return carry
    lax.fori_loop(0, 8, body, 0)

  def _store_rows(buf, g, lo, sz):
    def body(r8, carry):
      rowv = jnp.broadcast_to(r8, (16,)).astype(jnp.int32)
      tbase = g * 8 * _LP + r8 * _LP
      for i in range(4):
        t = tokflat[pl.ds(tbase + i * 16, 16)]
        wf = wfstage[pl.ds(r8 * _LP + i * 16, 16)]
        m = (t >= lo) & (t < lo + sz)
        plsc.store_scatter(buf, [rowv, t - lo], wf, mask=m)
      return carry
    lax.fori_loop(0, 8, body, 0)

  descs = {}
  for g in range(_NS):
    def _row(r8, carry):
      off0 = g * 8 * _LP + r8 * _LP
      ts, ws = [], []
      for i in range(4):
        t_i = tokflat[pl.ds(off0 + i * 16, 16)]
        w_i = wflat[pl.ds(off0 + i * 16, 16)]
        if i == 3:
          w_i = jnp.where(padmask, w_i, 0.0)
        tb[i, :] = t_i
        wb[i, :] = w_i
        ts.append(t_i)
        ws.append(w_i)

      def _rot(s, cacc):
        idxs = (lane + s) & 15
        cacc = list(cacc)
        for j in range(4):
          bj = jnp.broadcast_to(jnp.int32(j), (16,))
          tsv = plsc.load_gather(tb, [bj, idxs])
          wsv = plsc.load_gather(wb, [bj, idxs])
          for i in range(4):
            cacc[i] = cacc[i] + jnp.where(ts[i] == tsv, wsv, 0.0)
        return tuple(cacc)

      cs = lax.fori_loop(0, 16, _rot, (zvec, zvec, zvec, zvec))

      acc = cs[0] * ws[0] + cs[1] * ws[1] + cs[2] * ws[2] + cs[3] * ws[3]
      normsq = jnp.sum(acc)
      nv = jnp.broadcast_to(normsq, (16,))
      sq = nv * _rsqrt_vec(jnp.maximum(nv, 1e-37))
      scale = 1.0 / (sq + 1e-10)
      for i in range(4):
        wfstage[pl.ds(r8 * _LP + i * 16, 16)] = cs[i] * scale
      return carry
    lax.fori_loop(0, 8, _row, 0)

    rowbase = pl.multiple_of(wid * _RPW + g * 8, 8)
    for c in range(_NCH + 1):
      buf, sem, lo, sz = _chunk_buf(c)
      name = "T" if c == _NCH else ("A" if c % 2 == 0 else "B")
      if name in descs:
        descs[name].wait()
        if c == _NCH:
          prev_g, prev_c = g - 1, _NCH
        elif c >= 2:
          prev_g, prev_c = g, c - 2
        else:
          prev_g, prev_c = g - 1, c + _NCH - 2
        _reset_rows(buf, prev_g, _chunk_buf(prev_c)[2], _chunk_buf(prev_c)[3])
      _store_rows(buf, g, lo, sz)
      descs[name] = pltpu.async_copy(
          buf, out_hbm.at[pl.ds(rowbase, 8), pl.ds(lo, sz)], sem)

  for d in descs.values():
    d.wait()


_sc_bow = functools.partial(
    pl.kernel,
    out_type=jax.ShapeDtypeStruct((_B, _V), jnp.float32),
    mesh=plsc.VectorSubcoreMesh(core_axis_name="c", subcore_axis_name="s"),
    compiler_params=pltpu.CompilerParams(needs_layout_passes=False),
    scratch_types=[
        pltpu.VMEM((_TOKROWS, 128), jnp.int32),
        pltpu.VMEM((_RPW * _LP,), jnp.int32),
        pltpu.VMEM((_TOKROWS, 128), jnp.float32),
        pltpu.VMEM((_RPW * _LP,), jnp.float32),
        pltpu.VMEM((8 * _LP,), jnp.float32),
        pltpu.VMEM((4, 16), jnp.int32),
        pltpu.VMEM((4, 16), jnp.float32),
        pltpu.VMEM((8, _CCH), jnp.float32),
        pltpu.VMEM((8, _CCH), jnp.float32),
        pltpu.VMEM((8, _TAIL), jnp.float32),
        pltpu.SemaphoreType.DMA,
        pltpu.SemaphoreType.DMA,
        pltpu.SemaphoreType.DMA,
        pltpu.SemaphoreType.DMA,
    ],
)(_sc_body)


@jax.jit
def kernel(tokens, idf):
  B, L = tokens.shape
  (V,) = idf.shape
  tokens = tokens.astype(jnp.int32)
  pad = jnp.broadcast_to(tokens[:, :1], (B, _LP - L))
  tokp = jnp.concatenate([tokens, pad], axis=1)
  tok3d = tokp.reshape(_NW, _TOKROWS, 128)
  tokfl = tokp.reshape(_NW, _RPW * _LP)
  return _sc_bow(tok3d, tokfl, idf.astype(jnp.float32))
